# Initial kernel scaffold; baseline (speedup 1.0000x reference)
import jax
import jax.numpy as jnp
from jax.experimental import pallas as pl


def kernel(im_node_states, im_adj_matrix, sg_node_states, sg_edge_states, bbox, params, im_batch_list, sg_adj_list, sg_batch_list, sg_edge_batch_list):
    raise NotImplementedError("write your pallas kernel here")


# speedup vs baseline: 3.4587x; 3.4587x over previous
import jax
import jax.numpy as jnp
from jax import lax
from jax.experimental import pallas as pl

F32 = jnp.float32
N_SG = 512
N_IM = 512
E_SG = 4096
B = 8
ET = 16
EB = E_SG // ET


def _sig(x):
    x = jnp.clip(x, -60.0, 60.0)
    return 1.0 / (1.0 + jnp.exp(-x))


def _dot(a, b):
    return jnp.dot(a, b, preferred_element_type=F32)


def _p1_body(a_ref, b_ref, o_ref):
    @pl.when(pl.program_id(0) == 0)
    def _():
        o_ref[...] = jnp.zeros_like(o_ref)

    o_ref[...] += _dot(a_ref[...], b_ref[...])


def _p2_body(imx0_ref, bobj_ref, adj_ref, wmsg_ref, bmsg_ref, wu1_ref, wu2_ref,
             bupd_ref, wg_ref, bg_ref, wgtr_ref, bgtr_ref, imb_ref, out_ref):
    im_x = imx0_ref[...] + bobj_ref[...]
    m = _dot(adj_ref[...], _dot(im_x, wmsg_ref[...]) + bmsg_ref[...])
    im_x2 = jnp.maximum(_dot(im_x, wu1_ref[...]) + _dot(m, wu2_ref[...])
                        + bupd_ref[...], 0.0)
    gate = _sig(jnp.sum(im_x2 * wg_ref[...], axis=1, keepdims=True) + bg_ref[...])
    gated = gate * (_dot(im_x2, wgtr_ref[...]) + bgtr_ref[...])
    iota_b = lax.broadcasted_iota(jnp.int32, (B, N_IM), 0).astype(F32)
    Pim = (iota_b == imb_ref[...]).astype(F32)
    out_ref[...] = _dot(Pim, gated)


def _p3_body(bbox_ref, wp1_ref, bp1_ref, bng_ref, bnb_ref, wp2_ref, bp2_ref,
             sgn_ref, wol1_ref, wol2_ref, bol_ref, sge_ref, wrl_ref, brl_ref,
             we_ref, wi_ref, wj_ref, wa_ref,
             sgx_ref, t_ref, xi_ref, xj_ref, xa_ref):
    h = _dot(bbox_ref[...], wp1_ref[...]) + bp1_ref[...]
    mu = jnp.mean(h, axis=0, keepdims=True)
    var = jnp.mean((h - mu) ** 2, axis=0, keepdims=True)
    h = (h - mu) * lax.rsqrt(var + 1e-5) * bng_ref[...] + bnb_ref[...]
    pos = jnp.maximum(_dot(h, wp2_ref[...]) + bp2_ref[...], 0.0)
    sg_x = (_dot(sgn_ref[...], wol1_ref[...]) + _dot(pos, wol2_ref[...])
            + bol_ref[...])
    sg_e = _dot(sge_ref[...], wrl_ref[...]) + brl_ref[...]
    sgx_ref[...] = sg_x
    t_ref[...] = _dot(sg_e, we_ref[...])
    xi_ref[...] = _dot(sg_x, wi_ref[...])
    xj_ref[...] = _dot(sg_x, wj_ref[...])
    xa_ref[...] = _dot(sg_x, wa_ref[...])


def _p4_body(srcc_ref, srcr_ref, dstc_ref, ebr_ref, srcf_ref, dstf_ref,
             t_ref, xi_ref, xj_ref, xa_ref, be_ref, bnm_ref, weg_ref, beg_ref,
             wetr_ref, betr_ref, macc_ref, eacc_ref, epool_ref):
    k = pl.program_id(0)
    srcc = srcc_ref[0]
    srcr = srcr_ref[0]
    dstc = dstc_ref[0]
    keys_c = srcc * 512.0 + dstc

    tg = jnp.zeros((EB, 128), F32)
    cnt = jnp.zeros((EB, 1), F32)
    for l in range(ET):
        kl = srcf_ref[l] * 512.0 + dstf_ref[l]
        Et = (keys_c == kl).astype(F32)
        tg += _dot(Et, t_ref[l * EB:(l + 1) * EB, :])
        cnt += jnp.sum(Et, axis=1, keepdims=True)

    iota_r = lax.broadcasted_iota(jnp.int32, (1, N_SG), 1).astype(F32)
    iota_c = lax.broadcasted_iota(jnp.int32, (N_SG, 1), 0).astype(F32)
    A_src = (srcc == iota_r).astype(F32)
    A_dst = (dstc == iota_r).astype(F32)
    A_srcT = (iota_c == srcr).astype(F32)

    xi_g = _dot(A_src, xi_ref[...])
    xj_g = _dot(A_dst, xj_ref[...])
    xa_g = _dot(A_dst, xa_ref[...])
    e_new = jnp.maximum(xi_g + xj_g + tg + be_ref[...], 0.0)

    w = 1.0 / cnt
    cm = _dot(A_srcT, w * (xa_g + bnm_ref[...]))
    ce = _dot(A_srcT, w * e_new)

    eg = _sig(jnp.sum(e_new * weg_ref[...], axis=1, keepdims=True) + beg_ref[...])
    gated = eg * (_dot(e_new, wetr_ref[...]) + betr_ref[...])
    iota_b = lax.broadcasted_iota(jnp.int32, (B, EB), 0).astype(F32)
    Pe = (iota_b == ebr_ref[0]).astype(F32)
    pool = _dot(Pe, gated)

    @pl.when(k == 0)
    def _():
        macc_ref[...] = jnp.zeros_like(macc_ref)
        eacc_ref[...] = jnp.zeros_like(eacc_ref)
        epool_ref[...] = jnp.zeros_like(epool_ref)

    macc_ref[...] += cm
    eacc_ref[...] += ce
    epool_ref[...] += pool


def _p5_body(sgx_ref, macc_ref, eacc_ref, wb_ref, wn1_ref, wn2_ref, bnu_ref,
             wsg_ref, bsg_ref, wsgtr_ref, bsgtr_ref, sgb_ref, impool_ref,
             epool_ref, we1a_ref, we1b_ref, be1_ref, we2_ref, be2_ref, out_ref):
    msg = macc_ref[...] + _dot(eacc_ref[...], wb_ref[...])
    sgx2 = jnp.maximum(_dot(sgx_ref[...], wn1_ref[...])
                       + _dot(msg, wn2_ref[...]) + bnu_ref[...], 0.0)
    ng = _sig(jnp.sum(sgx2 * wsg_ref[...], axis=1, keepdims=True) + bsg_ref[...])
    gated = ng * (_dot(sgx2, wsgtr_ref[...]) + bsgtr_ref[...])
    iota_b = lax.broadcasted_iota(jnp.int32, (B, N_SG), 0).astype(F32)
    Psg = (iota_b == sgb_ref[...]).astype(F32)
    sg_pool = _dot(Psg, gated) + epool_ref[...]
    h2 = jnp.maximum(_dot(impool_ref[...], we1a_ref[...])
                     + _dot(sg_pool, we1b_ref[...]) + be1_ref[...], 0.0)
    out_ref[...] = jnp.sum(h2 * we2_ref[...], axis=1, keepdims=True) + be2_ref[...]


def kernel(im_node_states, im_adj_matrix, sg_node_states, sg_edge_states, bbox,
           params, im_batch_list, sg_adj_list, sg_batch_list, sg_edge_batch_list):
    p = params
    r2 = lambda v: v.reshape(1, -1)

    w_objT = p['W_obj'].T
    im_x0 = pl.pallas_call(
        _p1_body,
        grid=(8,),
        in_specs=[pl.BlockSpec((N_IM, 512), lambda k: (0, k)),
                  pl.BlockSpec((512, 256), lambda k: (k, 0))],
        out_specs=pl.BlockSpec((N_IM, 256), lambda k: (0, 0)),
        out_shape=jax.ShapeDtypeStruct((N_IM, 256), F32),
    )(im_node_states, w_objT)

    im_pool = pl.pallas_call(
        _p2_body,
        out_shape=jax.ShapeDtypeStruct((B, 512), F32),
    )(im_x0, r2(p['b_obj']), im_adj_matrix, p['W_msg'].T, r2(p['b_msg']),
      p['W_upd'][:, :256].T, p['W_upd'][:, 256:].T, r2(p['b_upd']),
      r2(p['Wg_gate'][0]), p['bg_gate'].reshape(1, 1), p['Wg_tr'].T,
      r2(p['bg_tr']), im_batch_list.astype(F32).reshape(1, N_IM))

    D = 256
    sg_x, t, xi, xj, xa = pl.pallas_call(
        _p3_body,
        out_shape=[jax.ShapeDtypeStruct((N_SG, 256), F32),
                   jax.ShapeDtypeStruct((E_SG, 128), F32),
                   jax.ShapeDtypeStruct((N_SG, 128), F32),
                   jax.ShapeDtypeStruct((N_SG, 128), F32),
                   jax.ShapeDtypeStruct((N_SG, 256), F32)],
    )(bbox, p['W_p1'].T, r2(p['b_p1']), r2(p['bn_g']), r2(p['bn_b']),
      p['W_p2'].T, r2(p['b_p2']), sg_node_states,
      p['W_objlab'][:, :151].T, p['W_objlab'][:, 151:].T, r2(p['b_objlab']),
      sg_edge_states, p['W_rellab'].T, r2(p['b_rellab']),
      p['W_e'][:, 2 * D:].T, p['W_e'][:, :D].T, p['W_e'][:, D:2 * D].T,
      p['W_nm'][:, :D].T)

    src = sg_adj_list[:, 0].astype(F32)
    dst = sg_adj_list[:, 1].astype(F32)
    srcc = src.reshape(ET, EB, 1)
    srcr = src.reshape(ET, 1, EB)
    dstc = dst.reshape(ET, EB, 1)
    dstr = dst.reshape(ET, 1, EB)
    ebr = sg_edge_batch_list.astype(F32).reshape(ET, 1, EB)

    tile_c = pl.BlockSpec((1, EB, 1), lambda k: (k, 0, 0))
    tile_r = pl.BlockSpec((1, 1, EB), lambda k: (k, 0, 0))
    full_r = pl.BlockSpec((ET, 1, EB), lambda k: (0, 0, 0))

    def whole(shape):
        return pl.BlockSpec(shape, lambda k: tuple(0 for _ in shape))

    macc, eacc, e_pool = pl.pallas_call(
        _p4_body,
        grid=(ET,),
        in_specs=[tile_c, tile_r, tile_c, tile_r, full_r, full_r,
                  whole((E_SG, 128)), whole((N_SG, 128)), whole((N_SG, 128)),
                  whole((N_SG, 256)), whole((1, 128)), whole((1, 256)),
                  whole((1, 128)), whole((1, 1)), whole((128, 512)),
                  whole((1, 512))],
        out_specs=[pl.BlockSpec((N_SG, 256), lambda k: (0, 0)),
                   pl.BlockSpec((N_SG, 128), lambda k: (0, 0)),
                   pl.BlockSpec((B, 512), lambda k: (0, 0))],
        out_shape=[jax.ShapeDtypeStruct((N_SG, 256), F32),
                   jax.ShapeDtypeStruct((N_SG, 128), F32),
                   jax.ShapeDtypeStruct((B, 512), F32)],
    )(srcc, srcr, dstc, ebr, srcr, dstr, t, xi, xj, xa,
      r2(p['b_e']), r2(p['b_nm']), r2(p['We_gate'][0]),
      p['be_gate'].reshape(1, 1), p['We_tr'].T, r2(p['be_tr']))

    energy = pl.pallas_call(
        _p5_body,
        out_shape=jax.ShapeDtypeStruct((B, 1), F32),
    )(sg_x, macc, eacc, p['W_nm'][:, D:].T, p['W_nu'][:, :256].T,
      p['W_nu'][:, 256:].T, r2(p['b_nu']), r2(p['Wsg_gate'][0]),
      p['bsg_gate'].reshape(1, 1), p['Wsg_tr'].T, r2(p['bsg_tr']),
      sg_batch_list.astype(F32).reshape(1, N_SG), im_pool, e_pool,
      p['W_e1'][:, :512].T, p['W_e1'][:, 512:].T, r2(p['b_e1']),
      r2(p['W_e2'][0]), p['b_e2'].reshape(1, 1))

    return energy
